# baseline (speedup 1.0000x reference)
import jax
import jax.numpy as jnp
from jax import lax
from jax.experimental import pallas as pl
from jax.experimental.pallas import tpu as pltpu
from jax.experimental.pallas import tpu_sc as plsc

N = 10000
E = 320000
D = 128

NC = 2
NS = 16
NW = NC * NS

BLK = 128
BPW = 80
NBLK_PAD = NW * BPW
E_PAD = NBLK_PAD * BLK

N_ACC = 10240
IHALF = 40
ZROWS = 32
OWN = N_ACC // NS


def _seg_sum_body(x0, x1, x2, s0, d0, s1, d1, s2, d2, out_hbm,
                  acc, zbuf, sidx, didx, rows0, rows1,
                  isem, zsem, g0sem, g1sem, wsem):
    cid = lax.axis_index("c")
    sid = lax.axis_index("s")
    wid = sid * NC + cid
    own0 = sid * OWN

    zeros16 = jnp.zeros((16,), jnp.float32)
    for r in range(ZROWS):
        for c in range(8):
            zbuf[r, pl.ds(c * 16, 16)] = zeros16

    def zero_own():
        zd = [pltpu.async_copy(zbuf, acc.at[pl.ds(own0 + r * ZROWS, ZROWS), :],
                               zsem)
              for r in range(OWN // ZROWS)]
        for d in zd:
            d.wait()

    zero_own()
    plsc.subcore_barrier()

    def main_graph(x_hbm, src_hbm, dst_hbm):
        row0 = wid * BPW

        def fire(j, rows, sem):
            return pltpu.async_copy(x_hbm.at[sidx.at[j]], rows, sem)

        for p in range(BPW // IHALF):
            prow = row0 + p * IHALF
            di = pltpu.async_copy(src_hbm.at[pl.ds(prow, IHALF), :], sidx,
                                  isem)
            dj = pltpu.async_copy(dst_hbm.at[pl.ds(prow, IHALF), :], didx,
                                  isem)
            di.wait()
            dj.wait()

            fire(0, rows0, g0sem)
            fire(1, rows1, g1sem)

            def blk_body(t, _):
                b = 2 * t
                pltpu.make_async_copy(x_hbm.at[sidx.at[b]], rows0,
                                      g0sem).wait()
                pltpu.sync_copy(rows0, acc.at[didx.at[b]], add=True)

                @pl.when(b + 2 < IHALF)
                def _():
                    fire(b + 2, rows0, g0sem)

                pltpu.make_async_copy(x_hbm.at[sidx.at[b + 1]], rows1,
                                      g1sem).wait()
                pltpu.sync_copy(rows1, acc.at[didx.at[b + 1]], add=True)

                @pl.when(b + 3 < IHALF)
                def _():
                    fire(b + 3, rows1, g1sem)

                return _

            lax.fori_loop(0, IHALF // 2, blk_body, None)

    for g, (x_hbm, src_hbm, dst_hbm) in enumerate(
            ((x0, s0, d0), (x1, s1, d1), (x2, s2, d2))):
        main_graph(x_hbm, src_hbm, dst_hbm)
        plsc.subcore_barrier()
        pltpu.async_copy(acc.at[pl.ds(own0, OWN), :],
                         out_hbm.at[g, cid, pl.ds(own0, OWN), :],
                         wsem).wait()
        if g < 2:
            zero_own()
            plsc.subcore_barrier()


def _seg_sum_sc(xs, srcs, dsts):
    mesh = plsc.VectorSubcoreMesh(core_axis_name="c", subcore_axis_name="s")
    return pl.kernel(
        _seg_sum_body,
        out_type=jax.ShapeDtypeStruct((3, NC, N_ACC, D), jnp.float32),
        mesh=mesh,
        scratch_types=[
            pltpu.VMEM_SHARED((N_ACC, D), jnp.float32),
            pltpu.VMEM((ZROWS, D), jnp.float32),
            pltpu.VMEM((IHALF, BLK), jnp.int32),
            pltpu.VMEM((IHALF, BLK), jnp.int32),
            pltpu.VMEM((BLK, D), jnp.float32),
            pltpu.VMEM((BLK, D), jnp.float32),
            pltpu.SemaphoreType.DMA,
            pltpu.SemaphoreType.DMA,
            pltpu.SemaphoreType.DMA,
            pltpu.SemaphoreType.DMA,
            pltpu.SemaphoreType.DMA,
        ],
    )(xs[0], xs[1], xs[2], srcs[0], dsts[0], srcs[1], dsts[1],
      srcs[2], dsts[2])


ROWS_TC = 1000


def _gcn_tc_body(pp_ref, x_ref, w_ref, b_ref, o_ref):
    s = pp_ref[0] + pp_ref[1]
    h = jnp.dot(s, w_ref[...], preferred_element_type=jnp.float32)
    o_ref[...] = jnp.maximum(h + b_ref[...], 0.0) + x_ref[...]


def _gcn_tc(pp, x, w, b):
    grid = (N // ROWS_TC,)
    pp_spec = pl.BlockSpec((NC, ROWS_TC, D), lambda i: (0, i, 0))
    row_spec = pl.BlockSpec((ROWS_TC, D), lambda i: (i, 0))
    full_spec = pl.BlockSpec((D, D), lambda i: (0, 0))
    bias_spec = pl.BlockSpec((1, D), lambda i: (0, 0))
    return pl.pallas_call(
        _gcn_tc_body,
        grid=grid,
        in_specs=[pp_spec, row_spec, full_spec, bias_spec],
        out_specs=row_spec,
        out_shape=jax.ShapeDtypeStruct((N, D), jnp.float32),
    )(pp, x, w, b)


_PAD = E_PAD - E


def kernel(h_mf_new, h_bp_new, h_cc_new, MF_sim_Graph, BP_sim_Graph,
           CC_sim_Graph, W_mf, b_mf, W_bp, b_bp, W_cc, b_cc):
    pad_iota = jnp.arange(_PAD, dtype=jnp.int32)
    pad_src = (pad_iota * 131) % N
    pad_dst = N + pad_iota % (N_ACC - N)
    xs = (h_mf_new, h_bp_new, h_cc_new)
    srcs, dsts = [], []
    for edges in (MF_sim_Graph, BP_sim_Graph, CC_sim_Graph):
        e32 = edges.astype(jnp.int32)
        srcs.append(jnp.concatenate([e32[0], pad_src]).reshape(NBLK_PAD, BLK))
        dsts.append(jnp.concatenate([e32[1], pad_dst]).reshape(NBLK_PAD, BLK))
    p = _seg_sum_sc(xs, srcs, dsts)
    ws = (W_mf, W_bp, W_cc)
    bs = (b_mf, b_bp, b_cc)
    return tuple(_gcn_tc(p[g], xs[g], ws[g], bs[g].reshape(1, D))
                 for g in range(3))


# speedup vs baseline: 1.0845x; 1.0845x over previous
import functools

import jax
import jax.numpy as jnp
from jax import lax
from jax.experimental import pallas as pl
from jax.experimental.pallas import tpu as pltpu
from jax.experimental.pallas import tpu_sc as plsc

N = 10000
E = 320000
D = 128

NC = 2
NS = 16
NW = NC * NS

BLK = 128
BPW = 80
NBLK_PAD = NW * BPW
E_PAD = NBLK_PAD * BLK

N_ACC = 10240
IHALF = 40
ZROWS = 32
OWN = N_ACC // NS


def _seg_sum_body(x_hbm, src_hbm, dst_hbm, out_hbm,
                  acc, zbuf, sidx, didx, rows0, rows1,
                  isem, zsem, g0sem, g1sem, wsem):
    cid = lax.axis_index("c")
    sid = lax.axis_index("s")
    wid = sid * NC + cid

    own0 = sid * OWN
    zeros16 = jnp.zeros((16,), jnp.float32)
    for r in range(ZROWS):
        for c in range(8):
            zbuf[r, pl.ds(c * 16, 16)] = zeros16
    row0 = wid * BPW
    di = pltpu.async_copy(src_hbm.at[pl.ds(row0, IHALF), :], sidx, isem)
    dj = pltpu.async_copy(dst_hbm.at[pl.ds(row0, IHALF), :], didx, isem)
    zd = [pltpu.async_copy(zbuf, acc.at[pl.ds(own0 + r * ZROWS, ZROWS), :],
                           zsem)
          for r in range(OWN // ZROWS)]
    di.wait()
    dj.wait()
    for d in zd:
        d.wait()
    plsc.subcore_barrier()

    def fire(j, rows, sem):
        return pltpu.async_copy(x_hbm.at[sidx.at[j]], rows, sem)

    for p in range(BPW // IHALF):
        if p > 0:
            prow = row0 + p * IHALF
            di = pltpu.async_copy(src_hbm.at[pl.ds(prow, IHALF), :], sidx,
                                  isem)
            dj = pltpu.async_copy(dst_hbm.at[pl.ds(prow, IHALF), :], didx,
                                  isem)
            di.wait()
            dj.wait()

        fire(0, rows0, g0sem)
        fire(1, rows1, g1sem)

        def blk_body(t, _):
            b = 2 * t
            pltpu.make_async_copy(x_hbm.at[sidx.at[b]], rows0, g0sem).wait()
            pltpu.sync_copy(rows0, acc.at[didx.at[b]], add=True)

            @pl.when(b + 2 < IHALF)
            def _():
                fire(b + 2, rows0, g0sem)

            pltpu.make_async_copy(x_hbm.at[sidx.at[b + 1]], rows1,
                                  g1sem).wait()
            pltpu.sync_copy(rows1, acc.at[didx.at[b + 1]], add=True)

            @pl.when(b + 3 < IHALF)
            def _():
                fire(b + 3, rows1, g1sem)

            return _

        lax.fori_loop(0, IHALF // 2, blk_body, None)

    plsc.subcore_barrier()

    pltpu.async_copy(acc.at[pl.ds(own0, OWN), :],
                     out_hbm.at[cid, pl.ds(own0, OWN), :],
                     wsem).wait()


def _seg_sum_sc(x, src2, dst2):
    mesh = plsc.VectorSubcoreMesh(core_axis_name="c", subcore_axis_name="s")
    return pl.kernel(
        _seg_sum_body,
        out_type=jax.ShapeDtypeStruct((NC, N_ACC, D), jnp.float32),
        mesh=mesh,
        scratch_types=[
            pltpu.VMEM_SHARED((N_ACC, D), jnp.float32),
            pltpu.VMEM((ZROWS, D), jnp.float32),
            pltpu.VMEM((IHALF, BLK), jnp.int32),
            pltpu.VMEM((IHALF, BLK), jnp.int32),
            pltpu.VMEM((BLK, D), jnp.float32),
            pltpu.VMEM((BLK, D), jnp.float32),
            pltpu.SemaphoreType.DMA,
            pltpu.SemaphoreType.DMA,
            pltpu.SemaphoreType.DMA,
            pltpu.SemaphoreType.DMA,
            pltpu.SemaphoreType.DMA,
        ],
    )(x, src2, dst2)


ROWS_TC = 1000


def _gcn_tc_body(pp_ref, x_ref, w_ref, b_ref, o_ref):
    s = pp_ref[0] + pp_ref[1]
    h = jnp.dot(s, w_ref[...], preferred_element_type=jnp.float32)
    o_ref[...] = jnp.maximum(h + b_ref[...], 0.0) + x_ref[...]


def _gcn_tc(pp, x, w, b):
    grid = (N // ROWS_TC,)
    pp_spec = pl.BlockSpec((NC, ROWS_TC, D), lambda i: (0, i, 0))
    row_spec = pl.BlockSpec((ROWS_TC, D), lambda i: (i, 0))
    full_spec = pl.BlockSpec((D, D), lambda i: (0, 0))
    bias_spec = pl.BlockSpec((1, D), lambda i: (0, 0))
    return pl.pallas_call(
        _gcn_tc_body,
        grid=grid,
        in_specs=[pp_spec, row_spec, full_spec, bias_spec],
        out_specs=row_spec,
        out_shape=jax.ShapeDtypeStruct((N, D), jnp.float32),
    )(pp, x, w, b)


_PAD = E_PAD - E


def kernel(h_mf_new, h_bp_new, h_cc_new, MF_sim_Graph, BP_sim_Graph,
           CC_sim_Graph, W_mf, b_mf, W_bp, b_bp, W_cc, b_cc):
    pad_iota = jnp.arange(_PAD, dtype=jnp.int32)
    pad_src = (pad_iota * 131) % N
    pad_dst = N + pad_iota % (N_ACC - N)
    outs = []
    for x, edges, w, b in (
        (h_mf_new, MF_sim_Graph, W_mf, b_mf),
        (h_bp_new, BP_sim_Graph, W_bp, b_bp),
        (h_cc_new, CC_sim_Graph, W_cc, b_cc),
    ):
        e32 = edges.astype(jnp.int32)
        src2 = jnp.concatenate([e32[0], pad_src]).reshape(NBLK_PAD, BLK)
        dst2 = jnp.concatenate([e32[1], pad_dst]).reshape(NBLK_PAD, BLK)
        partials = _seg_sum_sc(x, src2, dst2)
        outs.append(_gcn_tc(partials, x, w, b.reshape(1, D)))
    return tuple(outs)
